# Initial kernel scaffold; baseline (speedup 1.0000x reference)
import functools

import jax
import jax.numpy as jnp
from jax import lax
from jax.experimental import pallas as pl
from jax.experimental.pallas import tpu as pltpu
from jax.experimental.pallas import tpu_sc as plsc

NSEG = 64
FEAT = 128
LANE = 16
NVEC = FEAT // LANE
NCORE = 2
NSUB = 16
NW = NCORE * NSUB
RCHUNK = 256

SIZES = (100000, 50000, 25000)


def _cfg(n):
    c = -(-n // NW)
    c = -(-c // 8) * 8
    nch = -(-c // RCHUNK)
    return c, nch


_CFGS = tuple(_cfg(n) for n in SIZES)


def _sc_body(x0, x1, x2, s0, s1, s2,
             ps0, pm0, pc0, ps1, pm1, pc1, ps2, pm2, pc2,
             buf0, buf1, sg0, sg1, sg2, lsum, lmax, lcnt, sem0, sem1):
    wid = lax.axis_index("c") * NSUB + lax.axis_index("s")
    bufs = (buf0, buf1)
    sems = (sem0, sem1)
    zero = jnp.zeros((LANE,), jnp.float32)
    ninf = jnp.full((LANE,), -jnp.inf, jnp.float32)
    one = jnp.ones((LANE,), jnp.float32)

    layers = (
        (x0, s0, sg0, ps0, pm0, pc0, SIZES[0], _CFGS[0]),
        (x1, s1, sg1, ps1, pm1, pc1, SIZES[1], _CFGS[1]),
        (x2, s2, sg2, ps2, pm2, pc2, SIZES[2], _CFGS[2]),
    )

    for (x, seg, segv, psum, pmax, pcnt, n, (c, nch)) in layers:
        def _init(i, _):
            for k in range(NVEC):
                lsum[i, pl.ds(LANE * k, LANE)] = zero
                lmax[i, pl.ds(LANE * k, LANE)] = ninf
            lcnt[i, :] = zero
            return 0
        lax.fori_loop(0, NSEG, _init, 0)

        start = wid * c
        cnt = jnp.minimum(c, n - start)
        sstart = jnp.minimum(start, n - c)
        pltpu.sync_copy(seg.at[pl.ds(sstart, c)], segv)

        cs_eff = [None, None]
        cps = [None, None]
        cs_eff[0] = jnp.minimum(start, n - RCHUNK)
        cps[0] = pltpu.async_copy(x.at[pl.ds(cs_eff[0], RCHUNK)], bufs[0], sems[0])

        first_seg = segv[start - sstart]
        carry = (first_seg, zero, tuple([zero] * NVEC), tuple([ninf] * NVEC))

        for j in range(nch):
            b = j % 2
            if j + 1 < nch:
                nb = (j + 1) % 2
                cs_eff[nb] = jnp.minimum(start + (j + 1) * RCHUNK, n - RCHUNK)
                cps[nb] = pltpu.async_copy(
                    x.at[pl.ds(cs_eff[nb], RCHUNK)], bufs[nb], sems[nb])
            cps[b].wait()
            buf = bufs[b]
            boff = start + j * RCHUNK - cs_eff[b]
            sidx = start + j * RCHUNK - sstart
            valid = jnp.clip(cnt - j * RCHUNK, 0, RCHUNK)

            def _row(r, carry, buf=buf, boff=boff, sidx=sidx):
                cur, cv, accs, accm = carry
                s = segv[sidx + r]
                changed = s != cur

                @pl.when(changed)
                def _():
                    for k in range(NVEC):
                        lsum[cur, pl.ds(LANE * k, LANE)] = accs[k]
                        lmax[cur, pl.ds(LANE * k, LANE)] = accm[k]
                    lcnt[cur, :] = cv

                rows = [buf[boff + r, pl.ds(LANE * k, LANE)] for k in range(NVEC)]
                naccs = tuple(
                    jnp.where(changed, rows[k], accs[k] + rows[k])
                    for k in range(NVEC))
                naccm = tuple(
                    jnp.where(changed, rows[k], jnp.maximum(accm[k], rows[k]))
                    for k in range(NVEC))
                ncv = jnp.where(changed, one, cv + one)
                return (s, ncv, naccs, naccm)

            carry = lax.fori_loop(0, valid, _row, carry)

        cur, cv, accs, accm = carry
        for k in range(NVEC):
            lsum[cur, pl.ds(LANE * k, LANE)] = accs[k]
            lmax[cur, pl.ds(LANE * k, LANE)] = accm[k]
        lcnt[cur, :] = cv

        pltpu.sync_copy(lsum, psum.at[wid])
        pltpu.sync_copy(lmax, pmax.at[wid])
        pltpu.sync_copy(lcnt, pcnt.at[wid])


def _make_sc_call():
    mesh = plsc.VectorSubcoreMesh(core_axis_name="c", subcore_axis_name="s")
    f32 = jnp.float32
    outs = []
    for _n in SIZES:
        outs += [jax.ShapeDtypeStruct((NW, NSEG, FEAT), f32),
                 jax.ShapeDtypeStruct((NW, NSEG, FEAT), f32),
                 jax.ShapeDtypeStruct((NW, NSEG, LANE), f32)]
    scratch = [
        pltpu.VMEM((RCHUNK, FEAT), f32),
        pltpu.VMEM((RCHUNK, FEAT), f32),
        pltpu.VMEM((_CFGS[0][0],), jnp.int32),
        pltpu.VMEM((_CFGS[1][0],), jnp.int32),
        pltpu.VMEM((_CFGS[2][0],), jnp.int32),
        pltpu.VMEM((NSEG, FEAT), f32),
        pltpu.VMEM((NSEG, FEAT), f32),
        pltpu.VMEM((NSEG, LANE), f32),
        pltpu.SemaphoreType.DMA,
        pltpu.SemaphoreType.DMA,
    ]
    return pl.kernel(_sc_body, mesh=mesh, out_type=outs, scratch_types=scratch)


_sc_partials = _make_sc_call()


def _merge_body(ps0, pm0, pc0, ps1, pm1, pc1, ps2, pm2, pc2, out_ref):
    parts = ((ps0, pm0, pc0), (ps1, pm1, pc1), (ps2, pm2, pc2))
    for i, (ps, pm, pc) in enumerate(parts):
        s = jnp.sum(ps[...], axis=0)
        m = jnp.max(pm[...], axis=0)
        cvec = jnp.sum(pc[...], axis=0)
        cnt = cvec[:, 0:1]
        mean = s / jnp.maximum(cnt, 1.0)
        m = jnp.where(cnt > 0.0, m, 0.0)
        out_ref[:, i * FEAT:(i + 1) * FEAT] = m
        out_ref[:, 3 * FEAT + i * FEAT:3 * FEAT + (i + 1) * FEAT] = s
        out_ref[:, 6 * FEAT + i * FEAT:6 * FEAT + (i + 1) * FEAT] = mean


_merge = pl.pallas_call(
    _merge_body,
    out_shape=jax.ShapeDtypeStruct((NSEG, 9 * FEAT), jnp.float32),
)


def kernel(hs_0, hs_1, hs_2, gi_0, gi_1, gi_2):
    parts = _sc_partials(hs_0, hs_1, hs_2, gi_2, gi_1, gi_0)
    return _merge(*parts)


# speedup vs baseline: 6.5183x; 6.5183x over previous
import functools

import jax
import jax.numpy as jnp
from jax import lax
from jax.experimental import pallas as pl
from jax.experimental.pallas import tpu as pltpu
from jax.experimental.pallas import tpu_sc as plsc

NSEG = 64
FEAT = 128
LANE = 16
NVEC = FEAT // LANE
NCORE = 2
NSUB = 16
NW = NCORE * NSUB
RCHUNK = 256

SIZES = (100000, 50000, 25000)


def _cfg(n):
    c = -(-n // NW)
    c = -(-c // 8) * 8
    nch = -(-c // RCHUNK)
    return c, nch


_CFGS = tuple(_cfg(n) for n in SIZES)


def _sc_body(x0, x1, x2, s0, s1, s2,
             ps0, pm0, pc0, ps1, pm1, pc1, ps2, pm2, pc2,
             buf0, buf1, sg0, sg1, sg2, lsum, lmax, lcnt, sem0, sem1):
    wid = lax.axis_index("c") * NSUB + lax.axis_index("s")
    bufs = (buf0, buf1)
    sems = (sem0, sem1)
    zero = jnp.zeros((LANE,), jnp.float32)
    ninf = jnp.full((LANE,), -jnp.inf, jnp.float32)
    one = jnp.ones((LANE,), jnp.float32)

    layers = (
        (x0, s0, sg0, ps0, pm0, pc0, SIZES[0], _CFGS[0]),
        (x1, s1, sg1, ps1, pm1, pc1, SIZES[1], _CFGS[1]),
        (x2, s2, sg2, ps2, pm2, pc2, SIZES[2], _CFGS[2]),
    )

    for (x, seg, segv, psum, pmax, pcnt, n, (c, nch)) in layers:
        def _init(i, _):
            lsum[pl.ds(i * LANE, LANE)] = zero
            lmax[pl.ds(i * LANE, LANE)] = ninf
            return 0
        lax.fori_loop(0, NSEG * NVEC, _init, 0)

        def _initc(i, _):
            lcnt[pl.ds(i * LANE, LANE)] = zero
            return 0
        lax.fori_loop(0, NSEG, _initc, 0)

        start = wid * c
        cnt = jnp.minimum(c, n - start)
        sstart = jnp.minimum(start, n - c)
        pltpu.sync_copy(seg.at[pl.ds(sstart, c)], segv.at[pl.ds(0, c)])

        cs_eff = [None, None]
        cps = [None, None]
        cs_eff[0] = jnp.minimum(start, n - RCHUNK)
        cps[0] = pltpu.async_copy(
            x.at[pl.ds(cs_eff[0] * FEAT, RCHUNK * FEAT)], bufs[0], sems[0])

        first_seg = segv[pl.ds(start - sstart, LANE)][0]
        carry = (first_seg, zero, tuple([zero] * NVEC), tuple([ninf] * NVEC))

        for j in range(nch):
            b = j % 2
            if j + 1 < nch:
                nb = (j + 1) % 2
                cs_eff[nb] = jnp.minimum(start + (j + 1) * RCHUNK, n - RCHUNK)
                cps[nb] = pltpu.async_copy(
                    x.at[pl.ds(cs_eff[nb] * FEAT, RCHUNK * FEAT)],
                    bufs[nb], sems[nb])
            cps[b].wait()
            buf = bufs[b]
            boff = start + j * RCHUNK - cs_eff[b]
            sidx = start + j * RCHUNK - sstart
            valid = jnp.clip(cnt - j * RCHUNK, 0, RCHUNK)

            def _row(r, carry, buf=buf, boff=boff, sidx=sidx):
                cur, cv, accs, accm = carry
                s = segv[pl.ds(sidx + r, LANE)][0]
                changed = s != cur

                @pl.when(changed)
                def _():
                    for k in range(NVEC):
                        lsum[pl.ds(cur * FEAT + LANE * k, LANE)] = accs[k]
                        lmax[pl.ds(cur * FEAT + LANE * k, LANE)] = accm[k]
                    lcnt[pl.ds(cur * LANE, LANE)] = cv

                base = (boff + r) * FEAT
                rows = [buf[pl.ds(base + LANE * k, LANE)] for k in range(NVEC)]
                naccs = tuple(
                    jnp.where(changed, rows[k], accs[k] + rows[k])
                    for k in range(NVEC))
                naccm = tuple(
                    jnp.where(changed, rows[k], jnp.maximum(accm[k], rows[k]))
                    for k in range(NVEC))
                ncv = jnp.where(changed, one, cv + one)
                return (s, ncv, naccs, naccm)

            carry = lax.fori_loop(0, valid, _row, carry)

        cur, cv, accs, accm = carry
        for k in range(NVEC):
            lsum[pl.ds(cur * FEAT + LANE * k, LANE)] = accs[k]
            lmax[pl.ds(cur * FEAT + LANE * k, LANE)] = accm[k]
        lcnt[pl.ds(cur * LANE, LANE)] = cv

        pltpu.sync_copy(lsum, psum.at[wid])
        pltpu.sync_copy(lmax, pmax.at[wid])
        pltpu.sync_copy(lcnt, pcnt.at[wid])


@functools.cache
def _make_sc_call():
    mesh = plsc.VectorSubcoreMesh(core_axis_name="c", subcore_axis_name="s")
    f32 = jnp.float32
    outs = []
    for _n in SIZES:
        outs += [jax.ShapeDtypeStruct((NW, NSEG * FEAT), f32),
                 jax.ShapeDtypeStruct((NW, NSEG * FEAT), f32),
                 jax.ShapeDtypeStruct((NW, NSEG * LANE), f32)]
    scratch = [
        pltpu.VMEM((RCHUNK * FEAT,), f32),
        pltpu.VMEM((RCHUNK * FEAT,), f32),
        pltpu.VMEM((_CFGS[0][0] + LANE,), jnp.int32),
        pltpu.VMEM((_CFGS[1][0] + LANE,), jnp.int32),
        pltpu.VMEM((_CFGS[2][0] + LANE,), jnp.int32),
        pltpu.VMEM((NSEG * FEAT,), f32),
        pltpu.VMEM((NSEG * FEAT,), f32),
        pltpu.VMEM((NSEG * LANE,), f32),
        pltpu.SemaphoreType.DMA,
        pltpu.SemaphoreType.DMA,
    ]
    return pl.kernel(_sc_body, mesh=mesh, out_type=outs, scratch_types=scratch)


def _merge_body(ps0, pm0, pc0, ps1, pm1, pc1, ps2, pm2, pc2, out_ref):
    parts = ((ps0, pm0, pc0), (ps1, pm1, pc1), (ps2, pm2, pc2))
    for i, (ps, pm, pc) in enumerate(parts):
        s = jnp.sum(ps[...], axis=0)
        m = jnp.max(pm[...], axis=0)
        cvec = jnp.sum(pc[...], axis=0)
        cnt = cvec[:, 0:1]
        mean = s / jnp.maximum(cnt, 1.0)
        m = jnp.where(cnt > 0.0, m, 0.0)
        out_ref[:, i * FEAT:(i + 1) * FEAT] = m
        out_ref[:, 3 * FEAT + i * FEAT:3 * FEAT + (i + 1) * FEAT] = s
        out_ref[:, 6 * FEAT + i * FEAT:6 * FEAT + (i + 1) * FEAT] = mean


_merge = pl.pallas_call(
    _merge_body,
    out_shape=jax.ShapeDtypeStruct((NSEG, 9 * FEAT), jnp.float32),
)


def kernel(hs_0, hs_1, hs_2, gi_0, gi_1, gi_2):
    parts = _make_sc_call()(
        hs_0.reshape(-1), hs_1.reshape(-1), hs_2.reshape(-1),
        gi_2, gi_1, gi_0)
    shaped = []
    for i, p in enumerate(parts):
        if i % 3 == 2:
            shaped.append(p.reshape(NW, NSEG, LANE))
        else:
            shaped.append(p.reshape(NW, NSEG, FEAT))
    return _merge(*shaped)


# speedup vs baseline: 8.8137x; 1.3521x over previous
import functools

import jax
import jax.numpy as jnp
from jax import lax
from jax.experimental import pallas as pl
from jax.experimental.pallas import tpu as pltpu
from jax.experimental.pallas import tpu_sc as plsc

NSEG = 64
FEAT = 128
LANE = 16
NVEC = FEAT // LANE
NCORE = 2
NSUB = 16
NW = NCORE * NSUB
RCHUNK = 256

SIZES = (100000, 50000, 25000)


def _cfg(n):
    c = -(-n // NW)
    c = -(-c // 8) * 8
    nch = -(-c // RCHUNK)
    return c, nch


_CFGS = tuple(_cfg(n) for n in SIZES)


def _sc_body(x0, x1, x2, s0, s1, s2,
             ps0, pm0, pc0, ps1, pm1, pc1, ps2, pm2, pc2,
             buf0, buf1, sg0, sg1, sg2, lsum, lmax, lcnt, accv, curs,
             sem0, sem1):
    wid = lax.axis_index("c") * NSUB + lax.axis_index("s")
    bufs = (buf0, buf1)
    sems = (sem0, sem1)
    zero = jnp.zeros((LANE,), jnp.float32)
    ninf = jnp.full((LANE,), -jnp.inf, jnp.float32)
    one = jnp.ones((LANE,), jnp.float32)

    layers = (
        (x0, s0, sg0, ps0, pm0, pc0, SIZES[0], _CFGS[0]),
        (x1, s1, sg1, ps1, pm1, pc1, SIZES[1], _CFGS[1]),
        (x2, s2, sg2, ps2, pm2, pc2, SIZES[2], _CFGS[2]),
    )

    NB = RCHUNK // LANE

    for (x, seg, segv, psum, pmax, pcnt, n, (c, nch)) in layers:
        def _init(i, _):
            lsum[pl.ds(i * LANE, LANE)] = zero
            lmax[pl.ds(i * LANE, LANE)] = ninf
            return 0
        lax.fori_loop(0, NSEG * NVEC, _init, 0)

        def _initc(i, _):
            lcnt[pl.ds(i * LANE, LANE)] = zero
            return 0
        lax.fori_loop(0, NSEG, _initc, 0)

        start = wid * c
        cnt = jnp.minimum(c, n - start)
        sstart = jnp.minimum(start, n - c)
        pltpu.sync_copy(seg.at[pl.ds(sstart, c)], segv.at[pl.ds(0, c)])

        def _ceff(j):
            return jnp.minimum(start + j * RCHUNK, n - RCHUNK)

        def _flush_reset(new_seg):
            cur = curs[0]
            for k in range(NVEC):
                lsum[pl.ds(cur * FEAT + LANE * k, LANE)] = \
                    accv[pl.ds(LANE * k, LANE)]
                lmax[pl.ds(cur * FEAT + LANE * k, LANE)] = \
                    accv[pl.ds(FEAT + LANE * k, LANE)]
                accv[pl.ds(LANE * k, LANE)] = zero
                accv[pl.ds(FEAT + LANE * k, LANE)] = ninf
            lcnt[pl.ds(cur * LANE, LANE)] = accv[pl.ds(2 * FEAT, LANE)]
            accv[pl.ds(2 * FEAT, LANE)] = zero
            curs[0] = new_seg

        for k in range(NVEC):
            accv[pl.ds(LANE * k, LANE)] = zero
            accv[pl.ds(FEAT + LANE * k, LANE)] = ninf
        accv[pl.ds(2 * FEAT, LANE)] = zero
        curs[0] = segv[pl.ds(start - sstart, LANE)][0]

        pltpu.async_copy(
            x.at[pl.ds(_ceff(0) * FEAT, RCHUNK * FEAT)], bufs[0], sems[0])

        npair = (nch + 1) // 2

        def _pair(jj, _, x=x, segv=segv, c=c, n=n, start=start,
                  cnt=cnt, sstart=sstart, nch=nch):
            for b in range(2):
                j = 2 * jj + b
                ce = _ceff(j)

                @pl.when(j < nch)
                def _():
                    pltpu.make_async_copy(
                        x.at[pl.ds(ce * FEAT, RCHUNK * FEAT)],
                        bufs[b], sems[b]).wait()

                cen = _ceff(j + 1)

                @pl.when(j + 1 < nch)
                def _():
                    pltpu.async_copy(
                        x.at[pl.ds(cen * FEAT, RCHUNK * FEAT)],
                        bufs[1 - b], sems[1 - b])

                buf = bufs[b]
                boff = start + j * RCHUNK - ce
                sidx0 = start + j * RCHUNK - sstart
                valid = jnp.clip(cnt - j * RCHUNK, 0, RCHUNK)

                def _blk(blk, _, buf=buf, boff=boff, sidx0=sidx0,
                         valid=valid):
                    rem = valid - blk * LANE
                    sldx = jnp.minimum(sidx0 + blk * LANE, c)
                    svec = segv[pl.ds(sldx, LANE)]
                    s = svec[0]
                    fast = jnp.logical_and(svec[LANE - 1] == s, rem >= LANE)

                    @pl.when(fast)
                    def _():
                        @pl.when(s != curs[0])
                        def _():
                            _flush_reset(s)

                        accs = [accv[pl.ds(LANE * k, LANE)]
                                for k in range(NVEC)]
                        accm = [accv[pl.ds(FEAT + LANE * k, LANE)]
                                for k in range(NVEC)]
                        base = (boff + blk * LANE) * FEAT
                        for i in range(LANE):
                            for k in range(NVEC):
                                r = buf[pl.ds(base + i * FEAT + k * LANE,
                                              LANE)]
                                accs[k] = accs[k] + r
                                accm[k] = jnp.maximum(accm[k], r)
                        for k in range(NVEC):
                            accv[pl.ds(LANE * k, LANE)] = accs[k]
                            accv[pl.ds(FEAT + LANE * k, LANE)] = accm[k]
                        accv[pl.ds(2 * FEAT, LANE)] = \
                            accv[pl.ds(2 * FEAT, LANE)] + jnp.float32(LANE)

                    @pl.when(jnp.logical_not(fast))
                    def _():
                        def _row(r, _):
                            sr = segv[pl.ds(sidx0 + blk * LANE + r,
                                            LANE)][0]

                            @pl.when(sr != curs[0])
                            def _():
                                _flush_reset(sr)

                            base = (boff + blk * LANE + r) * FEAT
                            for k in range(NVEC):
                                rv = buf[pl.ds(base + LANE * k, LANE)]
                                accv[pl.ds(LANE * k, LANE)] = \
                                    accv[pl.ds(LANE * k, LANE)] + rv
                                accv[pl.ds(FEAT + LANE * k, LANE)] = \
                                    jnp.maximum(
                                        accv[pl.ds(FEAT + LANE * k, LANE)],
                                        rv)
                            accv[pl.ds(2 * FEAT, LANE)] = \
                                accv[pl.ds(2 * FEAT, LANE)] + one
                            return 0

                        lax.fori_loop(0, jnp.clip(rem, 0, LANE), _row, 0)

                    return 0

                lax.fori_loop(0, NB, _blk, 0)
            return 0

        lax.fori_loop(0, npair, _pair, 0)

        cur = curs[0]
        for k in range(NVEC):
            lsum[pl.ds(cur * FEAT + LANE * k, LANE)] = \
                accv[pl.ds(LANE * k, LANE)]
            lmax[pl.ds(cur * FEAT + LANE * k, LANE)] = \
                accv[pl.ds(FEAT + LANE * k, LANE)]
        lcnt[pl.ds(cur * LANE, LANE)] = accv[pl.ds(2 * FEAT, LANE)]

        pltpu.sync_copy(lsum, psum.at[wid])
        pltpu.sync_copy(lmax, pmax.at[wid])
        pltpu.sync_copy(lcnt, pcnt.at[wid])


@functools.cache
def _make_sc_call():
    mesh = plsc.VectorSubcoreMesh(core_axis_name="c", subcore_axis_name="s")
    f32 = jnp.float32
    outs = []
    for _n in SIZES:
        outs += [jax.ShapeDtypeStruct((NW, NSEG * FEAT), f32),
                 jax.ShapeDtypeStruct((NW, NSEG * FEAT), f32),
                 jax.ShapeDtypeStruct((NW, NSEG * LANE), f32)]
    scratch = [
        pltpu.VMEM((RCHUNK * FEAT,), f32),
        pltpu.VMEM((RCHUNK * FEAT,), f32),
        pltpu.VMEM((_CFGS[0][0] + LANE,), jnp.int32),
        pltpu.VMEM((_CFGS[1][0] + LANE,), jnp.int32),
        pltpu.VMEM((_CFGS[2][0] + LANE,), jnp.int32),
        pltpu.VMEM((NSEG * FEAT,), f32),
        pltpu.VMEM((NSEG * FEAT,), f32),
        pltpu.VMEM((NSEG * LANE,), f32),
        pltpu.VMEM((2 * FEAT + LANE,), f32),
        pltpu.SMEM((1,), jnp.int32),
        pltpu.SemaphoreType.DMA,
        pltpu.SemaphoreType.DMA,
    ]
    return pl.kernel(_sc_body, mesh=mesh, out_type=outs, scratch_types=scratch)


def _merge_body(ps0, pm0, pc0, ps1, pm1, pc1, ps2, pm2, pc2, out_ref):
    parts = ((ps0, pm0, pc0), (ps1, pm1, pc1), (ps2, pm2, pc2))
    for i, (ps, pm, pc) in enumerate(parts):
        s = jnp.sum(ps[...], axis=0)
        m = jnp.max(pm[...], axis=0)
        cvec = jnp.sum(pc[...], axis=0)
        cnt = cvec[:, 0:1]
        mean = s / jnp.maximum(cnt, 1.0)
        m = jnp.where(cnt > 0.0, m, 0.0)
        out_ref[:, i * FEAT:(i + 1) * FEAT] = m
        out_ref[:, 3 * FEAT + i * FEAT:3 * FEAT + (i + 1) * FEAT] = s
        out_ref[:, 6 * FEAT + i * FEAT:6 * FEAT + (i + 1) * FEAT] = mean


_merge = pl.pallas_call(
    _merge_body,
    out_shape=jax.ShapeDtypeStruct((NSEG, 9 * FEAT), jnp.float32),
)


def kernel(hs_0, hs_1, hs_2, gi_0, gi_1, gi_2):
    parts = _make_sc_call()(
        hs_0.reshape(-1), hs_1.reshape(-1), hs_2.reshape(-1),
        gi_2, gi_1, gi_0)
    shaped = []
    for i, p in enumerate(parts):
        if i % 3 == 2:
            shaped.append(p.reshape(NW, NSEG, LANE))
        else:
            shaped.append(p.reshape(NW, NSEG, FEAT))
    return _merge(*shaped)
